# Initial kernel scaffold; baseline (speedup 1.0000x reference)
import jax
import jax.numpy as jnp
from jax.experimental import pallas as pl


def kernel(x, size, mask, viz, g1, b1, W_qkv, W_proj, b_proj, g2, b2, W_fc1, b_fc1, W_fc2, b_fc2, merge_threshold, prune_threshold):
    raise NotImplementedError("write your pallas kernel here")


# speedup vs baseline: 2.9182x; 2.9182x over previous
import functools

import jax
import jax.numpy as jnp
from jax import lax
from jax.experimental import pallas as pl
from jax.experimental.pallas import tpu as pltpu

B, T, C, H = 8, 576, 768, 12
HD = C // H
S = T // 2
SCALE = HD ** -0.5
NEG = float("-inf")


def _iota(shape, dim):
    return lax.broadcasted_iota(jnp.int32, shape, dim)


def _attn_merge_body(x_ref, wq_ref, wk_ref, wv_ref, wp_ref, bp_ref, g1_ref,
                     b1_ref, mth_ref, pth_ref,
                     x2_ref, size_ref, mask_ref, idx_ref, mmask_ref):
    x = x_ref[0]
    g1 = g1_ref[...]
    b1 = b1_ref[...]
    mth = mth_ref[0, 0]
    pth = pth_ref[0, 0]

    mu = jnp.mean(x, axis=-1, keepdims=True)
    xc = x - mu
    var = jnp.mean(xc * xc, axis=-1, keepdims=True)
    xn = xc / jnp.sqrt(var + 1e-5) * g1 + b1

    q = jnp.dot(xn, wq_ref[...], preferred_element_type=jnp.float32) * SCALE
    k = jnp.dot(xn, wk_ref[...], preferred_element_type=jnp.float32)
    v = jnp.dot(xn, wv_ref[...], preferred_element_type=jnp.float32)

    lane_c = _iota((T, C), 1) // HD
    o = jnp.zeros((T, C), jnp.float32)
    imp_acc = jnp.zeros((1, T), jnp.float32)
    for h in range(H):
        hm = (lane_c == h).astype(jnp.float32)
        qh = q * hm
        logits = lax.dot_general(qh, k, (((1,), (1,)), ((), ())),
                                 preferred_element_type=jnp.float32)
        mx = jnp.max(logits, axis=-1, keepdims=True)
        e = jnp.exp(logits - mx)
        probs = e / jnp.sum(e, axis=-1, keepdims=True)
        imp_acc = imp_acc + jnp.sum(probs, axis=0, keepdims=True)
        vh = v * hm
        o = o + jnp.dot(probs, vh, preferred_element_type=jnp.float32)
    xo = jnp.dot(o, wp_ref[...], preferred_element_type=jnp.float32) + bp_ref[...]
    imp_row = imp_acc * (1.0 / (H * T))

    rmat = ((_iota((C, HD), 0) % HD) == _iota((C, HD), 1)).astype(jnp.float32)
    metric = jnp.dot(k, rmat, preferred_element_type=jnp.float32) * (1.0 / H)
    nrm = jnp.sqrt(jnp.sum(metric * metric, axis=-1, keepdims=True))
    metric = metric / nrm
    mr = metric.reshape(S, 2, HD)
    a_m = mr[:, 0, :]
    b_m = mr[:, 1, :]
    scores = lax.dot_general(a_m, b_m, (((1,), (1,)), ((), ())),
                             preferred_element_type=jnp.float32)
    scores = jnp.where(_iota((S, S), 0) == 0, NEG, scores)

    node_max = jnp.max(scores, axis=-1, keepdims=True)
    mmask_col = (node_max > mth).astype(jnp.float32)
    cand = jnp.where(scores == node_max, _iota((S, S), 1), S)
    idx_col = jnp.min(cand, axis=-1, keepdims=True)

    ident = (_iota((S, S), 0) == _iota((S, S), 1)).astype(jnp.float32)

    def row_of(col):
        return jnp.sum(ident * col, axis=0, keepdims=True)

    def col_of(row):
        return jnp.sum(ident * row, axis=1, keepdims=True)

    ohT = (idx_col == _iota((S, S), 1)).astype(jnp.float32)
    ohmT = ohT * mmask_col
    counts_row = jnp.sum(ohmT, axis=0, keepdims=True)
    counts_col = col_of(counts_row)

    x1 = x + xo
    x1r = x1.reshape(S, 2, C)
    src_x = x1r[:, 0, :]
    dst_x = x1r[:, 1, :]
    add_dst = lax.dot_general(ohmT, src_x, (((0,), (0,)), ((), ())),
                              preferred_element_type=jnp.float32)
    dst_x_new = (dst_x + add_dst) / (1.0 + counts_col)
    x2_ref[0, 0:S, :] = src_x
    x2_ref[0, S:T, :] = dst_x_new

    e_even = (_iota((T, S), 0) == 2 * _iota((T, S), 1)).astype(jnp.float32)
    e_odd = (_iota((T, S), 0) == 2 * _iota((T, S), 1) + 1).astype(jnp.float32)
    ident_t = (_iota((T, T), 0) == _iota((T, T), 1)).astype(jnp.float32)
    imp_col = jnp.sum(ident_t * imp_row, axis=1, keepdims=True)
    src_sc_row = jnp.sum(e_even * imp_col, axis=0, keepdims=True)
    dst_sc_row = jnp.sum(e_odd * imp_col, axis=0, keepdims=True)
    src_sc_col = col_of(src_sc_row)
    merge_sc_col = src_sc_col * mmask_col
    mat = jnp.where(ohmT != 0.0, merge_sc_col, NEG)
    scmax_row = jnp.max(mat, axis=0, keepdims=True)
    dst_sc_new = jnp.maximum(dst_sc_row, scmax_row)

    mmask_row = row_of(mmask_col)
    mask_src = (1.0 - mmask_row) * (src_sc_row > pth).astype(jnp.float32)
    mask_dst = (dst_sc_new > pth).astype(jnp.float32)
    mask_ref[0, 0:1, :] = mask_src
    mask_ref[0, 1:2, :] = mask_dst
    size_ref[0, 0:1, :] = jnp.ones((1, S), jnp.float32)
    size_ref[0, 1:2, :] = 1.0 + counts_row

    idx_row = row_of(idx_col.astype(jnp.float32))
    idx_ref[0] = jnp.round(idx_row).astype(jnp.int32)
    mmask_ref[0] = mmask_row


def _viz_merge_body(idx_sm, mm_sm, viz_ref, out_ref):
    b = pl.program_id(0)
    vr = viz_ref[0].reshape(S, 2, T)
    out_ref[0, 0:S, :] = vr[:, 0, :]
    out_ref[0, S:T, :] = vr[:, 1, :]

    def body(i, _):
        j = idx_sm[b, i]

        @pl.when(mm_sm[b, i] == 1)
        def _():
            row = viz_ref[0, pl.ds(2 * i, 1), :]
            cur = out_ref[0, pl.ds(S + j, 1), :]
            out_ref[0, pl.ds(S + j, 1), :] = jnp.maximum(cur, row)
        return 0

    lax.fori_loop(0, S, body, 0)


def _mlp_body(x_ref, wf1_ref, bf1_ref, wf2_ref, bf2_ref, g2_ref, b2_ref,
              out_ref):
    x = x_ref[0]
    mu = jnp.mean(x, axis=-1, keepdims=True)
    xc = x - mu
    var = jnp.mean(xc * xc, axis=-1, keepdims=True)
    xn = xc / jnp.sqrt(var + 1e-5) * g2_ref[...] + b2_ref[...]
    h = jnp.dot(xn, wf1_ref[...], preferred_element_type=jnp.float32) + bf1_ref[...]
    h = h * 0.5 * (1.0 + lax.erf(h * (2.0 ** -0.5)))
    y = jnp.dot(h, wf2_ref[...], preferred_element_type=jnp.float32) + bf2_ref[...]
    out_ref[0] = x + y


def _full(shape):
    nd = len(shape)
    return pl.BlockSpec(shape, lambda b: (0,) * nd)


def _batched(shape):
    nd = len(shape)
    return pl.BlockSpec((1,) + shape[1:], lambda b: (b,) + (0,) * (nd - 1))


def kernel(x, size, mask, viz, g1, b1, W_qkv, W_proj, b_proj, g2, b2,
           W_fc1, b_fc1, W_fc2, b_fc2, merge_threshold, prune_threshold):
    f32 = jnp.float32
    wq = W_qkv[:, 0:C]
    wk = W_qkv[:, C:2 * C]
    wv = W_qkv[:, 2 * C:3 * C]
    g1r = g1.reshape(1, C)
    b1r = b1.reshape(1, C)
    bpr = b_proj.reshape(1, C)
    g2r = g2.reshape(1, C)
    b2r = b2.reshape(1, C)
    bf1r = b_fc1.reshape(1, 4 * C)
    bf2r = b_fc2.reshape(1, C)
    mth = merge_threshold.reshape(1, 1)
    pth = prune_threshold.reshape(1, 1)

    x2, size2, mask2, idx, mmask = pl.pallas_call(
        _attn_merge_body,
        grid=(B,),
        in_specs=[
            _batched((B, T, C)),
            _full((C, C)), _full((C, C)), _full((C, C)), _full((C, C)),
            _full((1, C)), _full((1, C)), _full((1, C)),
            _full((1, 1)), _full((1, 1)),
        ],
        out_specs=[
            _batched((B, T, C)),
            _batched((B, 2, S)),
            _batched((B, 2, S)),
            _batched((B, 1, S)),
            _batched((B, 1, S)),
        ],
        out_shape=[
            jax.ShapeDtypeStruct((B, T, C), f32),
            jax.ShapeDtypeStruct((B, 2, S), f32),
            jax.ShapeDtypeStruct((B, 2, S), f32),
            jax.ShapeDtypeStruct((B, 1, S), jnp.int32),
            jax.ShapeDtypeStruct((B, 1, S), f32),
        ],
    )(x, wq, wk, wv, W_proj, bpr, g1r, b1r, mth, pth)

    idx_flat = idx.reshape(B, S)
    mm_flat = mmask.reshape(B, S).astype(jnp.int32)

    viz_out = pl.pallas_call(
        _viz_merge_body,
        grid_spec=pltpu.PrefetchScalarGridSpec(
            num_scalar_prefetch=2,
            grid=(B,),
            in_specs=[pl.BlockSpec((1, T, T), lambda b, i_sm, m_sm: (b, 0, 0))],
            out_specs=pl.BlockSpec((1, T, T), lambda b, i_sm, m_sm: (b, 0, 0)),
        ),
        out_shape=jax.ShapeDtypeStruct((B, T, T), f32),
    )(idx_flat, mm_flat, viz)

    x_out = pl.pallas_call(
        _mlp_body,
        grid=(B,),
        in_specs=[
            _batched((B, T, C)),
            _full((C, 4 * C)), _full((1, 4 * C)),
            _full((4 * C, C)), _full((1, C)),
            _full((1, C)), _full((1, C)),
        ],
        out_specs=_batched((B, T, C)),
        out_shape=jax.ShapeDtypeStruct((B, T, C), f32),
    )(x2, W_fc1, bf1r, W_fc2, bf2r, g2r, b2r)

    size_out = size2.reshape(B, T, 1)
    mask_out = mask2.reshape(B, T)
    return x_out, size_out, mask_out, viz_out
